# baseline (speedup 1.0000x reference)
import jax
import jax.numpy as jnp
from jax import lax
from jax.experimental import pallas as pl
from jax.experimental.pallas import tpu as pltpu
from jax.experimental.pallas import tpu_sc as plsc

import functools

ROWS = 128
COLS = 32768
K = 64
L = 16
NC = 2
NS = 16
NW = NC * NS
RPW = ROWS // NW
NB = 4096
BSHIFT = 32 - 12
BOFF = NB // 2
CAP = 4096
NCHUNK = COLS // L
MARGIN = 0.25


def _fkey(v):
    b = lax.bitcast_convert_type(v, jnp.int32)
    return b ^ (lax.shift_right_arithmetic(b, 31) & jnp.int32(0x7FFFFFFF))


def _ikey(key):
    f = key ^ (lax.shift_right_arithmetic(key, 31) & jnp.int32(0x7FFFFFFF))
    return lax.bitcast_convert_type(f, jnp.float32)


def _tec_body(scores_hbm, out_hbm, rowa_v, rowb_v, out_v, hist_v, cval_v,
              cidx_v, pcnt_v, cflag_v, sema, semb, osem):
    c = lax.axis_index("c")
    s = lax.axis_index("s")
    wid = s * NC + c
    lanes = lax.iota(jnp.int32, L)
    lane0 = lanes == 0
    ones = jnp.ones((L,), jnp.int32)
    zi = jnp.zeros((L,), jnp.int32)
    zf = jnp.zeros((L,), jnp.float32)

    @plsc.parallel_loop(0, COLS // L, unroll=4)
    def zero_out(i):
        out_v[pl.ds(i * L, L)] = zf

    row0 = wid * RPW
    bufs = [(rowa_v, sema), (rowb_v, semb)]
    pltpu.async_copy(scores_hbm.at[row0], rowa_v, sema)

    def count_ge(cand, cnt, nch):
        @plsc.parallel_loop(0, nch, carry=zi)
        def acc(ch, acc_v):
            kv = _fkey(cval_v[pl.ds(ch * L, L)])
            ge = (kv >= cand) & ((ch * L + lanes) < cnt)
            return acc_v + ge.astype(jnp.int32)
        return jnp.sum(acc)

    def refine(prefix_init, nbits, cnt):
        nch = lax.div(cnt + (L - 1), jnp.int32(L))

        def bit_step(j, prefix):
            cand = prefix + lax.shift_left(jnp.int32(1), nbits - 1 - j)
            cge = count_ge(cand, cnt, nch)
            return jnp.where(cge >= K, cand, prefix)
        return lax.fori_loop(0, nbits, bit_step, prefix_init)

    def collect_pre(row_v, floor_v):
        @plsc.parallel_loop(0, NCHUNK, unroll=8)
        def flag_pass(i):
            v = row_v[pl.ds(i * L, L)]
            pc = plsc.all_reduce_population_count(v >= floor_v)
            plsc.store_scatter(pcnt_v, [zi + i], pc, mask=lane0)

        @plsc.parallel_loop(0, NCHUNK // L, unroll=2, carry=zi)
        def fcomp(g, fcnt_v):
            pcs = pcnt_v[pl.ds(g * L, L)]
            m = pcs > 0
            pos = plsc.cumsum(m.astype(jnp.int32))
            tgt = fcnt_v + pos - 1
            plsc.store_scatter(cflag_v, [tgt], g * L + lanes, mask=m)
            return fcnt_v + plsc.all_reduce_population_count(m)
        return jnp.max(fcomp)

    def collect_gather(row_v, floor_v, nflag):
        @plsc.parallel_loop(0, nflag, carry=zi)
        def coll(ch, cnt_v):
            fid = cflag_v[pl.ds(ch, L)][0]
            v = row_v[pl.ds(fid * L, L)]
            m = (v >= floor_v) & (cnt_v <= CAP - 2 * L)
            pos = plsc.cumsum(m.astype(jnp.int32))
            tgt = cnt_v + pos - 1
            plsc.store_scatter(cval_v, [tgt], v, mask=m)
            plsc.store_scatter(cidx_v, [tgt], fid * L + lanes, mask=m)
            return cnt_v + plsc.all_reduce_population_count(m)
        return jnp.max(coll)

    def collect(row_v, floor_v):
        return collect_gather(row_v, floor_v, collect_pre(row_v, floor_v))

    def full_select(row_v):
        @plsc.parallel_loop(0, NB // L, unroll=4)
        def zero_hist(i):
            hist_v[pl.ds(i * L, L)] = zi

        @plsc.parallel_loop(0, COLS // L, unroll=4)
        def hist_pass(i):
            key = _fkey(row_v[pl.ds(i * L, L)])
            bucket = lax.shift_right_arithmetic(key, BSHIFT) + BOFF
            plsc.addupdate_scatter(hist_v, [bucket], ones)

        def scan_cond(carry):
            j, total, b_star, found = carry
            return (found == 0) & (j < NB // L)

        def scan_step(carry):
            j, total, b_star, found = carry
            base = NB - (j + 1) * L
            h = hist_v[pl.ds(base, L)]
            hr = lax.rev(h, (0,))
            cs = plsc.cumsum(hr)
            hit = (total + cs) >= K
            anyhit = jnp.any(hit)
            ffs = jnp.int32(L) - jnp.sum(hit.astype(jnp.int32))
            nb_id = base + (L - 1) - ffs
            b_star = jnp.where(anyhit, nb_id, b_star)
            found = jnp.where(anyhit, 1, found)
            return j + 1, total + jnp.sum(h), b_star, found
        _, _, b_star, _ = lax.while_loop(
            scan_cond, scan_step,
            (jnp.int32(0), jnp.int32(0), jnp.int32(0), jnp.int32(0)))

        prefix0 = lax.shift_left(b_star - BOFF, BSHIFT)
        cnt = collect(row_v, _ikey(zi + prefix0))
        t_key = refine(prefix0, BSHIFT, cnt)
        return t_key, cnt

    def spec_select(cnt):
        nch = lax.div(cnt + (L - 1), jnp.int32(L))
        nonneg = count_ge(jnp.int32(0), cnt, nch)
        prefix_init = jnp.where(nonneg >= K, jnp.int32(0),
                                jnp.int32(-2**31))
        return refine(prefix_init, 31, cnt)

    def restore_zeros(cnt):
        nch = lax.div(cnt + (L - 1), jnp.int32(L))

        def unscat(ch, carry):
            sl = pl.ds(ch * L, L)
            iv = cidx_v[sl]
            m = (ch * L + lanes) < cnt
            plsc.store_scatter(out_v, [iv], zf, mask=m)
            return carry
        lax.fori_loop(0, nch, unscat, 0)

    f_v = zf
    cnt_prev = jnp.int32(0)
    for r in range(RPW):
        row_v, sem = bufs[r % 2]
        pltpu.make_async_copy(scores_hbm.at[row0 + r], row_v, sem).wait()
        if r + 1 < RPW:
            nrow_v, nsem = bufs[(r + 1) % 2]
            pltpu.async_copy(scores_hbm.at[row0 + r + 1], nrow_v, nsem)

        if r == 0:
            t_key, cnt = full_select(row_v)
        else:
            nflag = collect_pre(row_v, f_v)
            pltpu.make_async_copy(out_v, out_hbm.at[row0 + r - 1],
                                  osem).wait()
            restore_zeros(cnt_prev)
            cnt_s = collect_gather(row_v, f_v, nflag)
            ok = (cnt_s >= K) & (cnt_s <= CAP - 2 * L)
            t_key, cnt = lax.cond(
                ok,
                lambda: (spec_select(cnt_s), cnt_s),
                lambda: full_select(row_v))

        t_val_v = _ikey(zi + t_key)
        nch = lax.div(cnt + (L - 1), jnp.int32(L))

        def scat(ch, carry):
            sl = pl.ds(ch * L, L)
            vv = cval_v[sl]
            iv = cidx_v[sl]
            m = (vv >= t_val_v) & ((ch * L + lanes) < cnt)
            plsc.store_scatter(out_v, [iv], vv, mask=m)
            return carry
        lax.fori_loop(0, nch, scat, 0)

        pltpu.async_copy(out_v, out_hbm.at[row0 + r], osem)
        cnt_prev = cnt

        f_v = t_val_v - jnp.float32(MARGIN)

    pltpu.make_async_copy(out_v, out_hbm.at[row0 + RPW - 1], osem).wait()


@functools.partial(
    pl.kernel,
    out_type=jax.ShapeDtypeStruct((ROWS, COLS), jnp.float32),
    mesh=plsc.VectorSubcoreMesh(core_axis_name="c", subcore_axis_name="s"),
    compiler_params=pltpu.CompilerParams(needs_layout_passes=False),
    scratch_types=[
        pltpu.VMEM((COLS,), jnp.float32),
        pltpu.VMEM((COLS,), jnp.float32),
        pltpu.VMEM((COLS,), jnp.float32),
        pltpu.VMEM((NB,), jnp.int32),
        pltpu.VMEM((CAP,), jnp.float32),
        pltpu.VMEM((CAP,), jnp.int32),
        pltpu.VMEM((NCHUNK,), jnp.int32),
        pltpu.VMEM((NCHUNK + L,), jnp.int32),
        pltpu.SemaphoreType.DMA,
        pltpu.SemaphoreType.DMA,
        pltpu.SemaphoreType.DMA,
    ],
)
def _topk_mask_sc(scores_hbm, out_hbm, rowa_v, rowb_v, out_v, hist_v,
                  cval_v, cidx_v, pcnt_v, cflag_v, sema, semb, osem):
    _tec_body(scores_hbm, out_hbm, rowa_v, rowb_v, out_v, hist_v, cval_v,
              cidx_v, pcnt_v, cflag_v, sema, semb, osem)


def kernel(scores, k):
    del k
    return _topk_mask_sc(scores)


# speedup vs baseline: 1.0415x; 1.0415x over previous
import jax
import jax.numpy as jnp
from jax import lax
from jax.experimental import pallas as pl
from jax.experimental.pallas import tpu as pltpu
from jax.experimental.pallas import tpu_sc as plsc

import functools

ROWS = 128
COLS = 32768
K = 64
L = 16
NC = 2
NS = 16
NW = NC * NS
RPW = ROWS // NW
NB = 4096
BSHIFT = 32 - 12
BOFF = NB // 2
CAP = 4096
NCHUNK = COLS // L
MARGIN = 0.25


def _fkey(v):
    b = lax.bitcast_convert_type(v, jnp.int32)
    return b ^ (lax.shift_right_arithmetic(b, 31) & jnp.int32(0x7FFFFFFF))


def _ikey(key):
    f = key ^ (lax.shift_right_arithmetic(key, 31) & jnp.int32(0x7FFFFFFF))
    return lax.bitcast_convert_type(f, jnp.float32)


def _tec_body(scores_hbm, out_hbm, rowa_v, rowb_v, out_v, hist_v, cval_v,
              cidx_v, pcnt_v, cflag_v, sema, semb, osem):
    c = lax.axis_index("c")
    s = lax.axis_index("s")
    wid = s * NC + c
    lanes = lax.iota(jnp.int32, L)
    lane0 = lanes == 0
    ones = jnp.ones((L,), jnp.int32)
    zi = jnp.zeros((L,), jnp.int32)
    zf = jnp.zeros((L,), jnp.float32)

    row0 = wid * RPW
    bufs = [(rowa_v, sema), (rowb_v, semb)]
    pltpu.async_copy(scores_hbm.at[row0], rowa_v, sema)

    @plsc.parallel_loop(0, COLS // L, unroll=4)
    def zero_out(i):
        out_v[pl.ds(i * L, L)] = zf

    def count_ge(cand, cnt, nch):
        @plsc.parallel_loop(0, nch, carry=zi)
        def acc(ch, acc_v):
            kv = _fkey(cval_v[pl.ds(ch * L, L)])
            ge = (kv >= cand) & ((ch * L + lanes) < cnt)
            return acc_v + ge.astype(jnp.int32)
        return jnp.sum(acc)

    def refine(prefix_init, nbits, cnt):
        nch = lax.div(cnt + (L - 1), jnp.int32(L))

        def bit_step(j, prefix):
            cand = prefix + lax.shift_left(jnp.int32(1), nbits - 1 - j)
            cge = count_ge(cand, cnt, nch)
            return jnp.where(cge >= K, cand, prefix)
        return lax.fori_loop(0, nbits, bit_step, prefix_init)

    def collect_pre(row_v, floor_v):
        @plsc.parallel_loop(0, NCHUNK, unroll=4)
        def flag_pass(i):
            v = row_v[pl.ds(i * L, L)]
            pc = plsc.all_reduce_population_count(v >= floor_v)
            plsc.store_scatter(pcnt_v, [zi + i], pc, mask=lane0)

        @plsc.parallel_loop(0, NCHUNK // L, unroll=2, carry=zi)
        def fcomp(g, fcnt_v):
            pcs = pcnt_v[pl.ds(g * L, L)]
            m = pcs > 0
            pos = plsc.cumsum(m.astype(jnp.int32))
            tgt = fcnt_v + pos - 1
            plsc.store_scatter(cflag_v, [tgt], g * L + lanes, mask=m)
            return fcnt_v + plsc.all_reduce_population_count(m)
        return jnp.max(fcomp)

    def collect_gather(row_v, floor_v, nflag):
        @plsc.parallel_loop(0, nflag, carry=zi)
        def coll(ch, cnt_v):
            fid = cflag_v[pl.ds(ch, L)][0]
            v = row_v[pl.ds(fid * L, L)]
            m = (v >= floor_v) & (cnt_v <= CAP - 2 * L)
            pos = plsc.cumsum(m.astype(jnp.int32))
            tgt = cnt_v + pos - 1
            plsc.store_scatter(cval_v, [tgt], v, mask=m)
            plsc.store_scatter(cidx_v, [tgt], fid * L + lanes, mask=m)
            return cnt_v + plsc.all_reduce_population_count(m)
        return jnp.max(coll)

    def collect(row_v, floor_v):
        return collect_gather(row_v, floor_v, collect_pre(row_v, floor_v))

    def full_select(row_v):
        @plsc.parallel_loop(0, NB // L, unroll=4)
        def zero_hist(i):
            hist_v[pl.ds(i * L, L)] = zi

        @plsc.parallel_loop(0, COLS // L, unroll=4)
        def hist_pass(i):
            key = _fkey(row_v[pl.ds(i * L, L)])
            bucket = lax.shift_right_arithmetic(key, BSHIFT) + BOFF
            plsc.addupdate_scatter(hist_v, [bucket], ones)

        def scan_cond(carry):
            j, total, b_star, found = carry
            return (found == 0) & (j < NB // L)

        def scan_step(carry):
            j, total, b_star, found = carry
            base = NB - (j + 1) * L
            h = hist_v[pl.ds(base, L)]
            hr = lax.rev(h, (0,))
            cs = plsc.cumsum(hr)
            hit = (total + cs) >= K
            anyhit = jnp.any(hit)
            ffs = jnp.int32(L) - jnp.sum(hit.astype(jnp.int32))
            nb_id = base + (L - 1) - ffs
            b_star = jnp.where(anyhit, nb_id, b_star)
            found = jnp.where(anyhit, 1, found)
            return j + 1, total + jnp.sum(h), b_star, found
        _, _, b_star, _ = lax.while_loop(
            scan_cond, scan_step,
            (jnp.int32(0), jnp.int32(0), jnp.int32(0), jnp.int32(0)))

        prefix0 = lax.shift_left(b_star - BOFF, BSHIFT)
        cnt = collect(row_v, _ikey(zi + prefix0))
        t_key = refine(prefix0, BSHIFT, cnt)
        return t_key, cnt

    def spec_select(cnt):
        nch = lax.div(cnt + (L - 1), jnp.int32(L))
        nonneg = count_ge(jnp.int32(0), cnt, nch)
        prefix_init = jnp.where(nonneg >= K, jnp.int32(0),
                                jnp.int32(-2**31))
        return refine(prefix_init, 31, cnt)

    def restore_zeros(cnt):
        nch = lax.div(cnt + (L - 1), jnp.int32(L))

        def unscat(ch, carry):
            sl = pl.ds(ch * L, L)
            iv = cidx_v[sl]
            m = (ch * L + lanes) < cnt
            plsc.store_scatter(out_v, [iv], zf, mask=m)
            return carry
        lax.fori_loop(0, nch, unscat, 0)

    f_v = zf
    cnt_prev = jnp.int32(0)
    for r in range(RPW):
        row_v, sem = bufs[r % 2]
        pltpu.make_async_copy(scores_hbm.at[row0 + r], row_v, sem).wait()
        if r + 1 < RPW:
            nrow_v, nsem = bufs[(r + 1) % 2]
            pltpu.async_copy(scores_hbm.at[row0 + r + 1], nrow_v, nsem)

        if r == 0:
            t_key, cnt = full_select(row_v)
        else:
            nflag = collect_pre(row_v, f_v)
            pltpu.make_async_copy(out_v, out_hbm.at[row0 + r - 1],
                                  osem).wait()
            restore_zeros(cnt_prev)
            cnt_s = collect_gather(row_v, f_v, nflag)
            ok = (cnt_s >= K) & (cnt_s <= CAP - 2 * L)
            t_key, cnt = lax.cond(
                ok,
                lambda: (spec_select(cnt_s), cnt_s),
                lambda: full_select(row_v))

        t_val_v = _ikey(zi + t_key)
        nch = lax.div(cnt + (L - 1), jnp.int32(L))

        def scat(ch, carry):
            sl = pl.ds(ch * L, L)
            vv = cval_v[sl]
            iv = cidx_v[sl]
            m = (vv >= t_val_v) & ((ch * L + lanes) < cnt)
            plsc.store_scatter(out_v, [iv], vv, mask=m)
            return carry
        lax.fori_loop(0, nch, scat, 0)

        pltpu.async_copy(out_v, out_hbm.at[row0 + r], osem)
        cnt_prev = cnt

        f_v = t_val_v - jnp.float32(MARGIN)

    pltpu.make_async_copy(out_v, out_hbm.at[row0 + RPW - 1], osem).wait()


@functools.partial(
    pl.kernel,
    out_type=jax.ShapeDtypeStruct((ROWS, COLS), jnp.float32),
    mesh=plsc.VectorSubcoreMesh(core_axis_name="c", subcore_axis_name="s"),
    compiler_params=pltpu.CompilerParams(needs_layout_passes=False),
    scratch_types=[
        pltpu.VMEM((COLS,), jnp.float32),
        pltpu.VMEM((COLS,), jnp.float32),
        pltpu.VMEM((COLS,), jnp.float32),
        pltpu.VMEM((NB,), jnp.int32),
        pltpu.VMEM((CAP,), jnp.float32),
        pltpu.VMEM((CAP,), jnp.int32),
        pltpu.VMEM((NCHUNK,), jnp.int32),
        pltpu.VMEM((NCHUNK + L,), jnp.int32),
        pltpu.SemaphoreType.DMA,
        pltpu.SemaphoreType.DMA,
        pltpu.SemaphoreType.DMA,
    ],
)
def _topk_mask_sc(scores_hbm, out_hbm, rowa_v, rowb_v, out_v, hist_v,
                  cval_v, cidx_v, pcnt_v, cflag_v, sema, semb, osem):
    _tec_body(scores_hbm, out_hbm, rowa_v, rowb_v, out_v, hist_v, cval_v,
              cidx_v, pcnt_v, cflag_v, sema, semb, osem)


def kernel(scores, k):
    del k
    return _topk_mask_sc(scores)
